# Initial kernel scaffold; baseline (speedup 1.0000x reference)
import jax
import jax.numpy as jnp
from jax.experimental import pallas as pl


def kernel(cls_score, label, epoch):
    raise NotImplementedError("write your pallas kernel here")


# speedup vs baseline: 9.1353x; 9.1353x over previous
import functools
import math

import jax
import jax.numpy as jnp
from jax import lax
from jax.experimental import pallas as pl
from jax.experimental.pallas import tpu as pltpu
from jax.experimental.pallas import tpu_sc as plsc

GAMMA = 2.0
BALANCE_PARAM = 0.25
LOSS_WEIGHT = 1.0

B, C = 4096, 1000
CPAD = 1024
N = B * C
NPAD = B * CPAD
ROWS_PER_BLK = 128
GRID = B // ROWS_PER_BLK

K = math.ceil(B * C * (1.0 - 0.9))

NBINS = 4096


def _losses(s, lab_f):
    e = jnp.exp(-jnp.abs(s))
    sp = jnp.log1p(e)
    rel0 = jnp.maximum(s, 0.0)
    bce0 = rel0 + sp
    bce1 = rel0 - s + sp
    bce_t = jnp.where(lab_f > 0.5, bce1, bce0)
    bce_c = jnp.where(lab_f > 0.5, bce0, bce1)
    pt_t = jnp.exp(-bce_t)
    pt_c = jnp.exp(-bce_c)
    loss = (LOSS_WEIGHT * BALANCE_PARAM) * ((1.0 - pt_t) ** 2) * bce_t
    corr = (LOSS_WEIGHT * BALANCE_PARAM) * ((1.0 - pt_c) ** 2) * bce_c
    return loss, corr


def _ul_only(s, lab_f):
    e = jnp.exp(-jnp.abs(s))
    sp = jnp.log1p(e)
    bce0 = jnp.maximum(s, 0.0) + sp
    pt0 = jnp.exp(-bce0)
    loss0 = (LOSS_WEIGHT * BALANCE_PARAM) * ((1.0 - pt0) ** 2) * bce0
    return jnp.where(lab_f < 0.5, loss0, 0.0)


def _pass_a_body(score_ref, label_ref, ul_ref):
    s = score_ref[...]
    lab = jnp.clip(label_ref[...], 0, None).astype(jnp.float32)
    ul = _ul_only(s, lab)
    pad = jnp.zeros((ROWS_PER_BLK, CPAD - C), jnp.float32)
    ul_ref[...] = jnp.concatenate([ul, pad], axis=1)


def _pass_c_body(score_ref, label_ref, thr_ref, sl_ref, sm_ref):
    s = score_ref[...]
    lab = jnp.clip(label_ref[...], 0, None).astype(jnp.float32)
    loss, corr = _losses(s, lab)
    ul = jnp.where(lab < 0.5, loss, 0.0)
    thr = thr_ref[0, 0]
    mod = jnp.where(ul < thr, loss, corr)
    sl_ref[...] = jnp.full((1, 1, 128), jnp.sum(loss), jnp.float32)
    sm_ref[...] = jnp.full((1, 1, 128), jnp.sum(mod), jnp.float32)


def _make_sc_hist(prefix_shift, bin_shift, bin_mask):
    info = plsc.get_sparse_core_info()
    nw = info.num_cores * info.num_subcores
    per_w = NPAD // nw
    chunk = 4096
    nchunks = per_w // chunk

    mesh = plsc.VectorSubcoreMesh(core_axis_name="c", subcore_axis_name="s")

    @functools.partial(
        pl.kernel,
        mesh=mesh,
        compiler_params=pltpu.CompilerParams(needs_layout_passes=False),
        out_type=jax.ShapeDtypeStruct((nw, NBINS), jnp.int32),
        scratch_types=[
            pltpu.VMEM((chunk,), jnp.float32),
            pltpu.VMEM((16,), jnp.int32),
            pltpu.VMEM((16 * NBINS,), jnp.int32),
            pltpu.VMEM((NBINS,), jnp.int32),
        ],
    )
    def sc_hist(ul_hbm, target_hbm, out_hbm, buf_v, tgt_v, hist_v, out_v):
        wid = lax.axis_index("s") * info.num_cores + lax.axis_index("c")
        pltpu.sync_copy(target_hbm, tgt_v)
        target = tgt_v[...]

        zeros16 = jnp.zeros((16,), jnp.int32)

        def zero_body(i, _):
            hist_v[pl.ds(i * 16, 16)] = zeros16
            return 0

        lax.fori_loop(0, NBINS, zero_body, 0)

        lanebase = lax.iota(jnp.int32, 16) * NBINS
        ones16 = jnp.ones((16,), jnp.int32)

        def elem_body(i, _):
            v = buf_v[pl.ds(i * 16, 16)]
            bits = plsc.bitcast(v, jnp.int32)
            pref = lax.shift_right_logical(bits, prefix_shift)
            m = pref == target
            binv = jnp.bitwise_and(
                lax.shift_right_logical(bits, bin_shift), bin_mask)
            plsc.addupdate_scatter(hist_v, [lanebase + binv], ones16, mask=m)
            return 0

        def chunk_body(ci, _):
            base = wid * per_w + ci * chunk
            pltpu.sync_copy(ul_hbm.at[pl.ds(base, chunk)], buf_v)
            lax.fori_loop(0, chunk // 16, elem_body, 0)
            return 0

        lax.fori_loop(0, nchunks, chunk_body, 0)

        def red_body(c, _):
            acc = zeros16
            for j in range(16):
                acc = acc + hist_v[pl.ds(j * NBINS + c * 16, 16)]
            out_v[pl.ds(c * 16, 16)] = acc
            return 0

        lax.fori_loop(0, NBINS // 16, red_body, 0)
        pltpu.sync_copy(out_v, out_hbm.at[wid])

    return sc_hist


@functools.lru_cache(maxsize=1)
def _sc_passes():
    return (
        _make_sc_hist(31, 19, 0xFFF),
        _make_sc_hist(19, 7, 0xFFF),
        _make_sc_hist(7, 0, 0x7F),
    )


def _pick(parts, kk):
    hist = jnp.sum(parts, axis=0)
    ssum = jnp.cumsum(hist[::-1])[::-1]
    b = jnp.sum((ssum >= kk).astype(jnp.int32)) - 1
    above = ssum[b] - hist[b]
    return b, kk - above


def kernel(cls_score, label, epoch):
    label = label.astype(jnp.int32)

    ul_pad = pl.pallas_call(
        _pass_a_body,
        grid=(GRID,),
        in_specs=[
            pl.BlockSpec((ROWS_PER_BLK, C), lambda g: (g, 0)),
            pl.BlockSpec((ROWS_PER_BLK, C), lambda g: (g, 0)),
        ],
        out_specs=pl.BlockSpec((ROWS_PER_BLK, CPAD), lambda g: (g, 0)),
        out_shape=jax.ShapeDtypeStruct((B, CPAD), jnp.float32),
    )(cls_score, label)

    ul_flat = ul_pad.reshape(NPAD)
    sc1, sc2, sc3 = _sc_passes()

    t0 = jnp.zeros((16,), jnp.int32)
    h1 = sc1(ul_flat, t0)
    b1, k2 = _pick(h1, jnp.int32(K))

    h2 = sc2(ul_flat, jnp.broadcast_to(b1, (16,)).astype(jnp.int32))
    b2, k3 = _pick(h2, k2)

    pref3 = (b1 << 12) | b2
    h3 = sc3(ul_flat, jnp.broadcast_to(pref3, (16,)).astype(jnp.int32))
    b3, _ = _pick(h3, k3)

    t_bits = (pref3 << 7) | b3
    thr = lax.bitcast_convert_type(t_bits.astype(jnp.int32), jnp.float32)
    thr = thr.reshape(1, 1)

    sum_loss, sum_mod = pl.pallas_call(
        _pass_c_body,
        grid=(GRID,),
        in_specs=[
            pl.BlockSpec((ROWS_PER_BLK, C), lambda g: (g, 0)),
            pl.BlockSpec((ROWS_PER_BLK, C), lambda g: (g, 0)),
            pl.BlockSpec(memory_space=pltpu.SMEM),
        ],
        out_specs=[
            pl.BlockSpec((1, 1, 128), lambda g: (g, 0, 0)),
            pl.BlockSpec((1, 1, 128), lambda g: (g, 0, 0)),
        ],
        out_shape=[
            jax.ShapeDtypeStruct((GRID, 1, 128), jnp.float32),
            jax.ShapeDtypeStruct((GRID, 1, 128), jnp.float32),
        ],
    )(cls_score, label, thr)

    total_loss = jnp.sum(sum_loss[:, 0, 0])
    total_mod = jnp.sum(sum_mod[:, 0, 0])
    total = jnp.where(epoch == 0, total_loss, total_mod)
    return total / jnp.float32(N)


# speedup vs baseline: 12.2174x; 1.3374x over previous
import functools
import math

import jax
import jax.numpy as jnp
from jax import lax
from jax.experimental import pallas as pl
from jax.experimental.pallas import tpu as pltpu
from jax.experimental.pallas import tpu_sc as plsc

GAMMA = 2.0
BALANCE_PARAM = 0.25
LOSS_WEIGHT = 1.0

B, C = 4096, 1000
CPAD = 1024
N = B * C
NPAD = B * CPAD
ROWS_PER_BLK = 128
GRID = B // ROWS_PER_BLK

K = math.ceil(B * C * (1.0 - 0.9))

NBINS = 4096


def _losses(s, lab_f):
    e = jnp.exp(-jnp.abs(s))
    sp = jnp.log1p(e)
    rel0 = jnp.maximum(s, 0.0)
    bce0 = rel0 + sp
    bce1 = rel0 - s + sp
    bce_t = jnp.where(lab_f > 0.5, bce1, bce0)
    bce_c = jnp.where(lab_f > 0.5, bce0, bce1)
    pt_t = jnp.exp(-bce_t)
    pt_c = jnp.exp(-bce_c)
    loss = (LOSS_WEIGHT * BALANCE_PARAM) * ((1.0 - pt_t) ** 2) * bce_t
    corr = (LOSS_WEIGHT * BALANCE_PARAM) * ((1.0 - pt_c) ** 2) * bce_c
    return loss, corr


def _ul_only(s, lab_f):
    e = jnp.exp(-jnp.abs(s))
    sp = jnp.log1p(e)
    bce0 = jnp.maximum(s, 0.0) + sp
    pt0 = jnp.exp(-bce0)
    loss0 = (LOSS_WEIGHT * BALANCE_PARAM) * ((1.0 - pt0) ** 2) * bce0
    return jnp.where(lab_f < 0.5, loss0, 0.0)


def _pass_a_body(score_ref, label_ref, ul_ref):
    s = score_ref[...]
    lab = jnp.clip(label_ref[...], 0, None).astype(jnp.float32)
    ul = _ul_only(s, lab)
    pad = jnp.zeros((ROWS_PER_BLK, CPAD - C), jnp.float32)
    ul_ref[...] = jnp.concatenate([ul, pad], axis=1)


def _pass_c_body(score_ref, label_ref, thr_ref, sl_ref, sm_ref):
    s = score_ref[...]
    lab = jnp.clip(label_ref[...], 0, None).astype(jnp.float32)
    loss, corr = _losses(s, lab)
    ul = jnp.where(lab < 0.5, loss, 0.0)
    thr = thr_ref[0, 0]
    mod = jnp.where(ul < thr, loss, corr)
    sl_ref[...] = jnp.full((1, 1, 128), jnp.sum(loss), jnp.float32)
    sm_ref[...] = jnp.full((1, 1, 128), jnp.sum(mod), jnp.float32)


def _make_sc_hist(prefix_shift, bin_shift, bin_mask, use_mask):
    info = plsc.get_sparse_core_info()
    nw = info.num_cores * info.num_subcores
    per_w = NPAD // nw
    chunk = 4096
    nchunks = per_w // chunk

    mesh = plsc.VectorSubcoreMesh(core_axis_name="c", subcore_axis_name="s")

    @functools.partial(
        pl.kernel,
        mesh=mesh,
        compiler_params=pltpu.CompilerParams(needs_layout_passes=False),
        out_type=jax.ShapeDtypeStruct((nw, NBINS), jnp.int32),
        scratch_types=[
            pltpu.VMEM((chunk,), jnp.float32),
            pltpu.VMEM((chunk,), jnp.float32),
            pltpu.VMEM((16,), jnp.int32),
            pltpu.VMEM((16 * NBINS,), jnp.int32),
            pltpu.VMEM((NBINS,), jnp.int32),
            pltpu.SemaphoreType.DMA,
            pltpu.SemaphoreType.DMA,
        ],
    )
    def sc_hist(ul_hbm, target_hbm, out_hbm, buf0, buf1, tgt_v, hist_v,
                out_v, sem0, sem1):
        wid = lax.axis_index("s") * info.num_cores + lax.axis_index("c")
        base_w = wid * per_w
        pltpu.sync_copy(target_hbm, tgt_v)
        target = tgt_v[...]

        zeros16 = jnp.zeros((16,), jnp.int32)

        def start_copy(buf, sem, ci):
            pltpu.make_async_copy(
                ul_hbm.at[pl.ds(base_w + ci * chunk, chunk)], buf, sem
            ).start()

        def wait_copy(buf, sem):
            pltpu.make_async_copy(
                ul_hbm.at[pl.ds(0, chunk)], buf, sem).wait()

        start_copy(buf0, sem0, 0)
        start_copy(buf1, sem1, 1)

        def zero_body(i, _):
            for u in range(8):
                hist_v[pl.ds(i * 128 + u * 16, 16)] = zeros16
            return 0

        lax.fori_loop(0, NBINS // 8, zero_body, 0)

        lanebase = lax.iota(jnp.int32, 16) * NBINS
        ones16 = jnp.ones((16,), jnp.int32)

        def process(buf):
            def elem_body(i, _):
                for u in range(8):
                    v = buf[pl.ds(i * 128 + u * 16, 16)]
                    bits = plsc.bitcast(v, jnp.int32)
                    binv = jnp.bitwise_and(
                        lax.shift_right_logical(bits, bin_shift), bin_mask)
                    if use_mask:
                        pref = lax.shift_right_logical(bits, prefix_shift)
                        m = pref == target
                        plsc.addupdate_scatter(
                            hist_v, [lanebase + binv], ones16, mask=m)
                    else:
                        plsc.addupdate_scatter(
                            hist_v, [lanebase + binv], ones16)
                return 0

            lax.fori_loop(0, chunk // 128, elem_body, 0)

        def chunk_body(ci, _):
            nxt = jnp.minimum(2 * ci + 2, nchunks - 2)
            wait_copy(buf0, sem0)
            process(buf0)
            start_copy(buf0, sem0, nxt)
            wait_copy(buf1, sem1)
            process(buf1)
            start_copy(buf1, sem1, nxt + 1)
            return 0

        lax.fori_loop(0, nchunks // 2, chunk_body, 0)
        wait_copy(buf0, sem0)
        wait_copy(buf1, sem1)

        def red_body(c, _):
            acc = zeros16
            for j in range(16):
                acc = acc + hist_v[pl.ds(j * NBINS + c * 16, 16)]
            out_v[pl.ds(c * 16, 16)] = acc
            return 0

        lax.fori_loop(0, NBINS // 16, red_body, 0)
        pltpu.sync_copy(out_v, out_hbm.at[wid])

    return sc_hist


@functools.lru_cache(maxsize=1)
def _sc_passes():
    return (
        _make_sc_hist(31, 19, 0xFFF, False),
        _make_sc_hist(19, 7, 0xFFF, True),
        _make_sc_hist(7, 0, 0x7F, True),
    )


def _pick(parts, kk):
    hist = jnp.sum(parts, axis=0)
    ssum = jnp.cumsum(hist[::-1])[::-1]
    b = jnp.sum((ssum >= kk).astype(jnp.int32)) - 1
    above = ssum[b] - hist[b]
    return b, kk - above


def kernel(cls_score, label, epoch):
    label = label.astype(jnp.int32)

    ul_pad = pl.pallas_call(
        _pass_a_body,
        grid=(GRID,),
        in_specs=[
            pl.BlockSpec((ROWS_PER_BLK, C), lambda g: (g, 0)),
            pl.BlockSpec((ROWS_PER_BLK, C), lambda g: (g, 0)),
        ],
        out_specs=pl.BlockSpec((ROWS_PER_BLK, CPAD), lambda g: (g, 0)),
        out_shape=jax.ShapeDtypeStruct((B, CPAD), jnp.float32),
    )(cls_score, label)

    ul_flat = ul_pad.reshape(NPAD)
    sc1, sc2, sc3 = _sc_passes()

    t0 = jnp.zeros((16,), jnp.int32)
    h1 = sc1(ul_flat, t0)
    b1, k2 = _pick(h1, jnp.int32(K))

    h2 = sc2(ul_flat, jnp.broadcast_to(b1, (16,)).astype(jnp.int32))
    b2, k3 = _pick(h2, k2)

    pref3 = (b1 << 12) | b2
    h3 = sc3(ul_flat, jnp.broadcast_to(pref3, (16,)).astype(jnp.int32))
    b3, _ = _pick(h3, k3)

    t_bits = (pref3 << 7) | b3
    thr = lax.bitcast_convert_type(t_bits.astype(jnp.int32), jnp.float32)
    thr = thr.reshape(1, 1)

    sum_loss, sum_mod = pl.pallas_call(
        _pass_c_body,
        grid=(GRID,),
        in_specs=[
            pl.BlockSpec((ROWS_PER_BLK, C), lambda g: (g, 0)),
            pl.BlockSpec((ROWS_PER_BLK, C), lambda g: (g, 0)),
            pl.BlockSpec(memory_space=pltpu.SMEM),
        ],
        out_specs=[
            pl.BlockSpec((1, 1, 128), lambda g: (g, 0, 0)),
            pl.BlockSpec((1, 1, 128), lambda g: (g, 0, 0)),
        ],
        out_shape=[
            jax.ShapeDtypeStruct((GRID, 1, 128), jnp.float32),
            jax.ShapeDtypeStruct((GRID, 1, 128), jnp.float32),
        ],
    )(cls_score, label, thr)

    total_loss = jnp.sum(sum_loss[:, 0, 0])
    total_mod = jnp.sum(sum_mod[:, 0, 0])
    total = jnp.where(epoch == 0, total_loss, total_mod)
    return total / jnp.float32(N)


# speedup vs baseline: 18.9019x; 1.5471x over previous
import functools
import math

import jax
import jax.numpy as jnp
from jax import lax
from jax.experimental import pallas as pl
from jax.experimental.pallas import tpu as pltpu
from jax.experimental.pallas import tpu_sc as plsc

GAMMA = 2.0
BALANCE_PARAM = 0.25
LOSS_WEIGHT = 1.0

B, C = 4096, 1000
CPAD = 1024
N = B * C
NPAD = B * CPAD
ROWS_PER_BLK = 128
GRID = B // ROWS_PER_BLK

K = math.ceil(B * C * (1.0 - 0.9))

NBINS = 4096


def _losses(s, lab_f):
    e = jnp.exp(-jnp.abs(s))
    sp = jnp.log1p(e)
    rel0 = jnp.maximum(s, 0.0)
    bce0 = rel0 + sp
    bce1 = rel0 - s + sp
    bce_t = jnp.where(lab_f > 0.5, bce1, bce0)
    bce_c = jnp.where(lab_f > 0.5, bce0, bce1)
    pt_t = jnp.exp(-bce_t)
    pt_c = jnp.exp(-bce_c)
    loss = (LOSS_WEIGHT * BALANCE_PARAM) * ((1.0 - pt_t) ** 2) * bce_t
    corr = (LOSS_WEIGHT * BALANCE_PARAM) * ((1.0 - pt_c) ** 2) * bce_c
    return loss, corr


def _ul_only(s, lab_f):
    e = jnp.exp(-jnp.abs(s))
    sp = jnp.log1p(e)
    bce0 = jnp.maximum(s, 0.0) + sp
    pt0 = jnp.exp(-bce0)
    loss0 = (LOSS_WEIGHT * BALANCE_PARAM) * ((1.0 - pt0) ** 2) * bce0
    return jnp.where(lab_f < 0.5, loss0, 0.0)


def _pass_a_body(score_ref, label_ref, ul_ref):
    s = score_ref[...]
    lab = jnp.clip(label_ref[...], 0, None).astype(jnp.float32)
    ul = _ul_only(s, lab)
    pad = jnp.zeros((ROWS_PER_BLK, CPAD - C), jnp.float32)
    ul_ref[...] = jnp.concatenate([ul, pad], axis=1)


def _pass_c_body(score_ref, label_ref, thr_ref, sl_ref, sm_ref):
    s = score_ref[...]
    lab = jnp.clip(label_ref[...], 0, None).astype(jnp.float32)
    loss, corr = _losses(s, lab)
    ul = jnp.where(lab < 0.5, loss, 0.0)
    thr = thr_ref[0, 0]
    mod = jnp.where(ul < thr, loss, corr)
    sl_ref[...] = jnp.full((1, 1, 128), jnp.sum(loss), jnp.float32)
    sm_ref[...] = jnp.full((1, 1, 128), jnp.sum(mod), jnp.float32)


def _make_sc_hist(prefix_shift, bin_shift, bin_mask, use_mask):
    info = plsc.get_sparse_core_info()
    nw = info.num_cores * info.num_subcores
    per_w = NPAD // nw
    chunk = 4096
    nchunks = per_w // chunk

    mesh = plsc.VectorSubcoreMesh(core_axis_name="c", subcore_axis_name="s")

    @functools.partial(
        pl.kernel,
        mesh=mesh,
        compiler_params=pltpu.CompilerParams(needs_layout_passes=False),
        out_type=jax.ShapeDtypeStruct((nw, NBINS), jnp.int32),
        scratch_types=[
            pltpu.VMEM((chunk,), jnp.float32),
            pltpu.VMEM((chunk,), jnp.float32),
            pltpu.VMEM((16,), jnp.int32),
            pltpu.VMEM((16 * NBINS,), jnp.int32),
            pltpu.VMEM((NBINS,), jnp.int32),
            pltpu.SemaphoreType.DMA,
            pltpu.SemaphoreType.DMA,
        ],
    )
    def sc_hist(ul_hbm, target_hbm, out_hbm, buf0, buf1, tgt_v, hist_v,
                out_v, sem0, sem1):
        wid = lax.axis_index("s") * info.num_cores + lax.axis_index("c")
        base_w = wid * per_w
        pltpu.sync_copy(target_hbm, tgt_v)
        target = tgt_v[...]

        zeros16 = jnp.zeros((16,), jnp.int32)

        def start_copy(buf, sem, ci):
            pltpu.make_async_copy(
                ul_hbm.at[pl.ds(base_w + ci * chunk, chunk)], buf, sem
            ).start()

        def wait_copy(buf, sem):
            pltpu.make_async_copy(
                ul_hbm.at[pl.ds(0, chunk)], buf, sem).wait()

        start_copy(buf0, sem0, 0)
        start_copy(buf1, sem1, 1)

        @plsc.parallel_loop(0, NBINS // 16, unroll=8)
        def _(i):
            hist_v[pl.ds(i * 16, 16)] = zeros16

        lanebase = lax.iota(jnp.int32, 16) * NBINS
        ones16 = jnp.ones((16,), jnp.int32)

        def process(buf):
            @plsc.parallel_loop(0, chunk // 16, unroll=8)
            def _(i):
                v = buf[pl.ds(i * 16, 16)]
                bits = plsc.bitcast(v, jnp.int32)
                binv = jnp.bitwise_and(
                    lax.shift_right_logical(bits, bin_shift), bin_mask)
                if use_mask:
                    pref = lax.shift_right_logical(bits, prefix_shift)
                    m = pref == target
                    plsc.addupdate_scatter(
                        hist_v, [lanebase + binv], ones16, mask=m)
                else:
                    plsc.addupdate_scatter(
                        hist_v, [lanebase + binv], ones16)

        def chunk_body(ci, _):
            nxt = jnp.minimum(2 * ci + 2, nchunks - 2)
            wait_copy(buf0, sem0)
            process(buf0)
            start_copy(buf0, sem0, nxt)
            wait_copy(buf1, sem1)
            process(buf1)
            start_copy(buf1, sem1, nxt + 1)
            return 0

        lax.fori_loop(0, nchunks // 2, chunk_body, 0)
        wait_copy(buf0, sem0)
        wait_copy(buf1, sem1)

        @plsc.parallel_loop(0, NBINS // 16, unroll=4)
        def _(c):
            acc = zeros16
            for j in range(16):
                acc = acc + hist_v[pl.ds(j * NBINS + c * 16, 16)]
            out_v[pl.ds(c * 16, 16)] = acc
        pltpu.sync_copy(out_v, out_hbm.at[wid])

    return sc_hist


@functools.lru_cache(maxsize=1)
def _sc_passes():
    return (
        _make_sc_hist(31, 19, 0xFFF, False),
        _make_sc_hist(19, 7, 0xFFF, True),
        _make_sc_hist(7, 0, 0x7F, True),
    )


def _pick(parts, kk):
    hist = jnp.sum(parts, axis=0)
    ssum = jnp.cumsum(hist[::-1])[::-1]
    b = jnp.sum((ssum >= kk).astype(jnp.int32)) - 1
    above = ssum[b] - hist[b]
    return b, kk - above


def kernel(cls_score, label, epoch):
    label = label.astype(jnp.int32)

    ul_pad = pl.pallas_call(
        _pass_a_body,
        grid=(GRID,),
        in_specs=[
            pl.BlockSpec((ROWS_PER_BLK, C), lambda g: (g, 0)),
            pl.BlockSpec((ROWS_PER_BLK, C), lambda g: (g, 0)),
        ],
        out_specs=pl.BlockSpec((ROWS_PER_BLK, CPAD), lambda g: (g, 0)),
        out_shape=jax.ShapeDtypeStruct((B, CPAD), jnp.float32),
    )(cls_score, label)

    ul_flat = ul_pad.reshape(NPAD)
    sc1, sc2, sc3 = _sc_passes()

    t0 = jnp.zeros((16,), jnp.int32)
    h1 = sc1(ul_flat, t0)
    b1, k2 = _pick(h1, jnp.int32(K))

    h2 = sc2(ul_flat, jnp.broadcast_to(b1, (16,)).astype(jnp.int32))
    b2, k3 = _pick(h2, k2)

    pref3 = (b1 << 12) | b2
    h3 = sc3(ul_flat, jnp.broadcast_to(pref3, (16,)).astype(jnp.int32))
    b3, _ = _pick(h3, k3)

    t_bits = (pref3 << 7) | b3
    thr = lax.bitcast_convert_type(t_bits.astype(jnp.int32), jnp.float32)
    thr = thr.reshape(1, 1)

    sum_loss, sum_mod = pl.pallas_call(
        _pass_c_body,
        grid=(GRID,),
        in_specs=[
            pl.BlockSpec((ROWS_PER_BLK, C), lambda g: (g, 0)),
            pl.BlockSpec((ROWS_PER_BLK, C), lambda g: (g, 0)),
            pl.BlockSpec(memory_space=pltpu.SMEM),
        ],
        out_specs=[
            pl.BlockSpec((1, 1, 128), lambda g: (g, 0, 0)),
            pl.BlockSpec((1, 1, 128), lambda g: (g, 0, 0)),
        ],
        out_shape=[
            jax.ShapeDtypeStruct((GRID, 1, 128), jnp.float32),
            jax.ShapeDtypeStruct((GRID, 1, 128), jnp.float32),
        ],
    )(cls_score, label, thr)

    total_loss = jnp.sum(sum_loss[:, 0, 0])
    total_mod = jnp.sum(sum_mod[:, 0, 0])
    total = jnp.where(epoch == 0, total_loss, total_mod)
    return total / jnp.float32(N)
